# baseline (speedup 1.0000x reference)
import functools

import jax
import jax.numpy as jnp
from jax import lax
from jax.experimental import pallas as pl
from jax.experimental.pallas import tpu as pltpu
from jax.experimental.pallas import tpu_sc as plsc

_VOCAB = 1000000
_DIM = 32
_BATCH = 16384

_NC = 2
_NS = 16
_NW = _NC * _NS
_B_PER_W = _BATCH // _NW
_GROUP = 16
_N_GROUPS = _B_PER_W // _GROUP

_mesh = plsc.VectorSubcoreMesh(core_axis_name="c", subcore_axis_name="s")


@functools.partial(
    pl.kernel,
    mesh=_mesh,
    out_type=jax.ShapeDtypeStruct((_BATCH, _DIM), jnp.float32),
    scratch_types=[
        pltpu.VMEM((_B_PER_W,), jnp.int32),
        pltpu.VMEM((_B_PER_W, _DIM), jnp.float32),
        pltpu.SemaphoreType.DMA,
    ],
    compiler_params=pltpu.CompilerParams(use_tc_tiling_on_sc=True),
)
def _gather_kernel(idx_hbm, table_hbm, out_hbm, idx_v, rows_v, sem):
    wid = lax.axis_index("s") * _NC + lax.axis_index("c")
    base = wid * _B_PER_W
    pltpu.sync_copy(idx_hbm.at[pl.ds(base, _B_PER_W)], idx_v)

    def body(g, carry):
        r0 = g * _GROUP
        idx16 = idx_v[pl.ds(r0, _GROUP)]
        for lane in range(_GROUP):
            i = idx16[lane]
            pltpu.async_copy(
                table_hbm.at[pl.ds(i, 1)],
                rows_v.at[pl.ds(r0 + lane, 1)],
                sem,
            )
        pltpu.make_async_copy(
            table_hbm.at[pl.ds(0, _GROUP)],
            rows_v.at[pl.ds(r0, _GROUP)],
            sem,
        ).wait()
        return carry

    lax.fori_loop(0, _N_GROUPS, body, 0)
    pltpu.sync_copy(rows_v, out_hbm.at[pl.ds(base, _B_PER_W)])


def kernel(nodes, ordered_embs):
    idx = jnp.reshape(nodes.astype(jnp.int32), (_BATCH,))
    return _gather_kernel(idx, ordered_embs)


# speedup vs baseline: 1.0319x; 1.0319x over previous
import functools

import jax
import jax.numpy as jnp
from jax import lax
from jax.experimental import pallas as pl
from jax.experimental.pallas import tpu as pltpu
from jax.experimental.pallas import tpu_sc as plsc

_VOCAB = 1000000
_DIM = 32
_BATCH = 16384

_NC = 2
_NS = 16
_NW = _NC * _NS
_B_PER_W = _BATCH // _NW
_GROUP = 16
_N_GROUPS = _B_PER_W // _GROUP
_L = 16

_mesh = plsc.VectorSubcoreMesh(core_axis_name="c", subcore_axis_name="s")


@functools.partial(
    pl.kernel,
    mesh=_mesh,
    out_type=jax.ShapeDtypeStruct((_DIM, _BATCH), jnp.float32),
    scratch_types=[
        pltpu.VMEM((_B_PER_W,), jnp.int32),
        pltpu.VMEM((_B_PER_W, _DIM), jnp.float32),
        pltpu.VMEM((_DIM, _B_PER_W), jnp.float32),
        pltpu.SemaphoreType.DMA,
    ],
    compiler_params=pltpu.CompilerParams(needs_layout_passes=False),
)
def _gather_kernel(idx_hbm, table_hbm, outT_hbm, idx_v, rows_v, colsT_v, sem):
    wid = lax.axis_index("s") * _NC + lax.axis_index("c")
    base = wid * _B_PER_W
    pltpu.sync_copy(idx_hbm.at[pl.ds(base, _B_PER_W)], idx_v)

    def issue_group(g):
        r0 = g * _GROUP
        idx16 = idx_v[pl.ds(r0, _GROUP)]
        for lane in range(_GROUP):
            i = idx16[lane]
            pltpu.async_copy(
                table_hbm.at[pl.ds(i, 1)],
                rows_v.at[pl.ds(r0 + lane, 1)],
                sem,
            )

    def drain_group(g):
        pltpu.make_async_copy(
            table_hbm.at[pl.ds(0, _GROUP)],
            rows_v.at[pl.ds(g * _GROUP, _GROUP)],
            sem,
        ).wait()

    issue_group(0)

    def body(g, carry):
        issue_group(g)
        drain_group(g - 1)
        return carry

    lax.fori_loop(1, _N_GROUPS, body, 0)
    drain_group(_N_GROUPS - 1)

    def tbody(r, carry):
        col = jnp.full((_L,), r, jnp.int32)
        for c0 in (0, 16):
            v = rows_v[r, pl.ds(c0, _L)]
            rowsel = lax.iota(jnp.int32, _L) + c0
            plsc.store_scatter(colsT_v, [rowsel, col], v)
        return carry

    lax.fori_loop(0, _B_PER_W, tbody, 0)

    o0 = pl.multiple_of(base, 128)
    pltpu.sync_copy(colsT_v, outT_hbm.at[:, pl.ds(o0, _B_PER_W)])


def kernel(nodes, ordered_embs):
    idx = jnp.reshape(nodes.astype(jnp.int32), (_BATCH,))
    outT = _gather_kernel(idx, ordered_embs)
    return outT.T
